# baseline (speedup 1.0000x reference)
import functools

import jax
import jax.numpy as jnp
from jax import lax
from jax.experimental import pallas as pl
from jax.experimental.pallas import tpu as pltpu
from jax.experimental.pallas import tpu_sc as plsc

_V = 8192
_SPLIT = 2
_ROWS = 2 * 8192
_D = _V // _SPLIT
_NW = 32
_R_PER_W = _ROWS // _NW
_K = 8
_NCHUNK = _R_PER_W // _K


def _make_gather():
  mesh = plsc.VectorSubcoreMesh(core_axis_name="c", subcore_axis_name="s")

  @functools.partial(
      pl.kernel,
      mesh=mesh,
      out_type=jax.ShapeDtypeStruct((_ROWS, _D), jnp.float32),
      scratch_types=[
          pltpu.VMEM((_R_PER_W,), jnp.int32),
          pltpu.VMEM((_K, _D), jnp.float32),
          pltpu.SemaphoreType.DMA,
      ],
  )
  def gather_kernel(idx_hbm, table_hbm, out_hbm, idx_v, rows_v, sem):
    wid = lax.axis_index("s") * 2 + lax.axis_index("c")
    base = wid * _R_PER_W
    pltpu.sync_copy(idx_hbm.at[pl.ds(base, _R_PER_W)], idx_v)

    def chunk(i, carry):
      off = i * _K
      pltpu.async_copy(
          table_hbm.at[idx_v.at[pl.ds(off, _K)]], rows_v, sem
      ).wait()
      pltpu.sync_copy(rows_v, out_hbm.at[pl.ds(base + off, _K)])
      return carry

    lax.fori_loop(0, _NCHUNK, chunk, 0)

  return gather_kernel


_gather = _make_gather()


def kernel(idx, logits_table):
  b, s = idx.shape
  flat = idx.reshape(-1).astype(jnp.int32)
  half_idx = (flat[:, None] * _SPLIT
              + jnp.arange(_SPLIT, dtype=jnp.int32)).reshape(-1)
  table2 = logits_table.reshape(_SPLIT * _V, _D)
  out = _gather(half_idx, table2)
  return out.reshape(b, s, _V)


# speedup vs baseline: 1.0268x; 1.0268x over previous
import functools

import jax
import jax.numpy as jnp
from jax import lax
from jax.experimental import pallas as pl
from jax.experimental.pallas import tpu as pltpu
from jax.experimental.pallas import tpu_sc as plsc

_V = 8192
_SPLIT = 4
_ROWS = _SPLIT * 8192
_D = _V // _SPLIT
_NW = 32
_R_PER_W = _ROWS // _NW
_K = 8
_NCHUNK = _R_PER_W // _K
_NBUF = 4


def _make_gather():
  mesh = plsc.VectorSubcoreMesh(core_axis_name="c", subcore_axis_name="s")

  @functools.partial(
      pl.kernel,
      mesh=mesh,
      out_type=jax.ShapeDtypeStruct((_ROWS, _D), jnp.float32),
      scratch_types=[
          pltpu.VMEM((_R_PER_W,), jnp.int32),
          pltpu.VMEM((_NBUF, _K, _D), jnp.float32),
          pltpu.SemaphoreType.DMA,
          pltpu.SemaphoreType.DMA,
      ],
  )
  def gather_kernel(idx_hbm, table_hbm, out_hbm, idx_v, rows_v, gsem, ssem):
    wid = lax.axis_index("s") * 2 + lax.axis_index("c")
    base = wid * _R_PER_W
    pltpu.sync_copy(idx_hbm.at[pl.ds(base, _R_PER_W)], idx_v)

    def start_gather(t, buf):
      pltpu.async_copy(
          table_hbm.at[idx_v.at[pl.ds(t * _K, _K)]], rows_v.at[buf], gsem)

    def start_scatter(t, buf):
      pltpu.async_copy(
          rows_v.at[buf], out_hbm.at[pl.ds(base + t * _K, _K)], ssem)

    def wait_gather(buf):
      pltpu.make_async_copy(
          table_hbm.at[pl.ds(0, _K)], rows_v.at[buf], gsem).wait()

    def wait_scatter(buf):
      pltpu.make_async_copy(
          rows_v.at[buf], out_hbm.at[pl.ds(base, _K)], ssem).wait()

    start_gather(0, 0)
    start_gather(1, 1)
    wait_gather(0)
    start_scatter(0, 0)
    start_gather(2, 2)
    wait_gather(1)
    start_scatter(1, 1)
    start_gather(3, 3)

    def block(j, carry):
      t0 = 2 + j * _NBUF
      for b in range(_NBUF):
        t = t0 + b
        b_next = b
        b_cur = (2 + b) % _NBUF
        wait_scatter(b_next)
        start_gather(t + 2, b_next)
        wait_gather(b_cur)
        start_scatter(t, b_cur)
      return carry

    lax.fori_loop(0, (_NCHUNK - 4) // _NBUF, block, 0)

    n = _NCHUNK
    wait_scatter((n - 4) % _NBUF)
    wait_gather((n - 2) % _NBUF)
    start_scatter(n - 2, (n - 2) % _NBUF)
    wait_scatter((n - 3) % _NBUF)
    wait_gather((n - 1) % _NBUF)
    start_scatter(n - 1, (n - 1) % _NBUF)
    wait_scatter((n - 2) % _NBUF)
    wait_scatter((n - 1) % _NBUF)

  return gather_kernel


_gather = _make_gather()


def kernel(idx, logits_table):
  b, s = idx.shape
  flat = idx.reshape(-1).astype(jnp.int32)
  split_idx = (flat[:, None] * _SPLIT
               + jnp.arange(_SPLIT, dtype=jnp.int32)).reshape(-1)
  table2 = logits_table.reshape(_SPLIT * _V, _D)
  out = _gather(split_idx, table2)
  return out.reshape(b, s, _V)


# speedup vs baseline: 3.9101x; 3.8080x over previous
import functools

import jax
import jax.numpy as jnp
from jax import lax
from jax.experimental import pallas as pl
from jax.experimental.pallas import tpu as pltpu
from jax.experimental.pallas import tpu_sc as plsc

_V = 8192
_ROWS = 8192
_NW = 32
_R_PER_W = _ROWS // _NW
_K = 4
_NCHUNK = _R_PER_W // _K
_NBUF = 3


def _make_gather():
  mesh = plsc.VectorSubcoreMesh(core_axis_name="c", subcore_axis_name="s")

  @functools.partial(
      pl.kernel,
      mesh=mesh,
      out_type=jax.ShapeDtypeStruct((_ROWS, _V), jnp.float32),
      scratch_types=[
          pltpu.VMEM((_NCHUNK, _K), jnp.int32),
          pltpu.VMEM((_NBUF, _K, _V), jnp.float32),
          pltpu.SemaphoreType.DMA,
          pltpu.SemaphoreType.DMA,
      ],
  )
  def gather_kernel(idx_hbm, table_hbm, out_hbm, idx_v, rows_v, gsem, ssem):
    wid = lax.axis_index("s") * 2 + lax.axis_index("c")
    base = wid * _R_PER_W
    pltpu.sync_copy(idx_hbm.at[wid], idx_v)

    def start_gather(t, buf):
      pltpu.async_copy(table_hbm.at[idx_v.at[t]], rows_v.at[buf], gsem)

    def start_scatter(t, buf):
      pltpu.async_copy(
          rows_v.at[buf], out_hbm.at[pl.ds(base + t * _K, _K)], ssem)

    def wait_gather(buf):
      pltpu.make_async_copy(
          table_hbm.at[pl.ds(0, _K)], rows_v.at[buf], gsem).wait()

    def wait_scatter(buf):
      pltpu.make_async_copy(
          rows_v.at[buf], out_hbm.at[pl.ds(base, _K)], ssem).wait()

    start_gather(0, 0)
    start_gather(1, 1)
    wait_gather(0)
    start_scatter(0, 0)
    start_gather(2, 2)
    wait_gather(1)
    start_scatter(1, 1)
    wait_scatter(0)
    start_gather(3, 0)

    def block(j, carry):
      t0 = 2 + j * _NBUF
      for b in range(_NBUF):
        t = t0 + b
        b_cur = (2 + b) % _NBUF
        b_next = (1 + b) % _NBUF
        wait_gather(b_cur)
        start_scatter(t, b_cur)
        wait_scatter(b_next)
        start_gather(t + 2, b_next)
      return carry

    lax.fori_loop(0, (_NCHUNK - 4) // _NBUF, block, 0)

    n = _NCHUNK
    wait_gather((n - 2) % _NBUF)
    start_scatter(n - 2, (n - 2) % _NBUF)
    wait_scatter((n - 3) % _NBUF)
    wait_gather((n - 1) % _NBUF)
    start_scatter(n - 1, (n - 1) % _NBUF)
    wait_scatter((n - 2) % _NBUF)
    wait_scatter((n - 1) % _NBUF)

  return gather_kernel


_gather = _make_gather()


def kernel(idx, logits_table):
  b, s = idx.shape
  idx3 = idx.reshape(_NW, _NCHUNK, _K).astype(jnp.int32)
  out = _gather(idx3, logits_table)
  return out.reshape(b, s, _V)
